# Initial kernel scaffold; baseline (speedup 1.0000x reference)
import jax
import jax.numpy as jnp
from jax.experimental import pallas as pl

_AXIS_RES = 512
_EMB = 256
_HID = 128
_FREQ = 30.0


def _fused_body(c_ref, e0_ref, e1_ref, w0_ref, b0_ref, w1_ref, b1_ref,
                w2_ref, b2_ref, o_ref):
    c = jnp.clip(c_ref[...], -1.0, 0.999)
    t = (0.5 * c + 0.5) * (_AXIS_RES - 1)
    bsz = c.shape[0]
    cols = jax.lax.broadcasted_iota(jnp.float32, (bsz, _AXIS_RES), 1)
    s0 = jnp.maximum(1.0 - jnp.abs(t[:, 0:1] - cols), 0.0)
    s1 = jnp.maximum(1.0 - jnp.abs(t[:, 1:2] - cols), 0.0)
    e0 = jnp.dot(s0, e0_ref[...], preferred_element_type=jnp.float32)
    e1 = jnp.dot(s1, e1_ref[...], preferred_element_type=jnp.float32)
    x = e0 * e1
    h = jnp.sin(_FREQ * (jnp.dot(x, w0_ref[...],
                                 preferred_element_type=jnp.float32) + b0_ref[...]))
    h = jnp.sin(_FREQ * (jnp.dot(h, w1_ref[...],
                                 preferred_element_type=jnp.float32) + b1_ref[...]))
    o_ref[...] = jnp.dot(h, w2_ref[...],
                         preferred_element_type=jnp.float32) + b2_ref[...]


def kernel(coords, emb0, emb1, W0, b0, W1, b1, W2, b2):
    n = coords.shape[0]
    bsz = 2048
    grid = (n // bsz,)
    rep = lambda i: (0, 0)
    out = pl.pallas_call(
        _fused_body,
        grid=grid,
        in_specs=[
            pl.BlockSpec((bsz, 2), lambda i: (i, 0)),
            pl.BlockSpec((_AXIS_RES, _EMB), rep),
            pl.BlockSpec((_AXIS_RES, _EMB), rep),
            pl.BlockSpec((_EMB, _HID), rep),
            pl.BlockSpec((1, _HID), rep),
            pl.BlockSpec((_HID, _HID), rep),
            pl.BlockSpec((1, _HID), rep),
            pl.BlockSpec((_HID, 3), rep),
            pl.BlockSpec((1, 3), rep),
        ],
        out_specs=pl.BlockSpec((bsz, 3), lambda i: (i, 0)),
        out_shape=jax.ShapeDtypeStruct((n, 3), jnp.float32),
    )(coords, emb0, emb1, W0.T, b0.reshape(1, -1), W1.T, b1.reshape(1, -1),
      W2.T, b2.reshape(1, -1))
    return out


# speedup vs baseline: 4.0979x; 4.0979x over previous
import jax
import jax.numpy as jnp
from jax.experimental import pallas as pl

_AXIS_RES = 512
_EMB = 256
_HID = 128
_FREQ = 30.0


def _fused_body(c_ref, e0_ref, e1_ref, w0_ref, b0_ref, w1_ref, b1_ref,
                w2_ref, b2_ref, o_ref):
    c = jnp.clip(c_ref[...], -1.0, 0.999)
    t = (0.5 * c + 0.5) * (_AXIS_RES - 1)
    bsz = c.shape[0]
    cols = jax.lax.broadcasted_iota(jnp.int32, (bsz, _AXIS_RES), 1).astype(jnp.float32)
    s0 = jnp.maximum(1.0 - jnp.abs(t[:, 0:1] - cols), 0.0)
    s1 = jnp.maximum(1.0 - jnp.abs(t[:, 1:2] - cols), 0.0)
    e0 = jnp.dot(s0, e0_ref[...], preferred_element_type=jnp.float32)
    e1 = jnp.dot(s1, e1_ref[...], preferred_element_type=jnp.float32)
    x = e0 * e1
    h = jnp.sin(_FREQ * (jnp.dot(x, w0_ref[...],
                                 preferred_element_type=jnp.float32) + b0_ref[...]))
    h = jnp.sin(_FREQ * (jnp.dot(h, w1_ref[...],
                                 preferred_element_type=jnp.float32) + b1_ref[...]))
    o_ref[...] = jnp.dot(h, w2_ref[...],
                         preferred_element_type=jnp.float32) + b2_ref[...]


def kernel(coords, emb0, emb1, W0, b0, W1, b1, W2, b2):
    n = coords.shape[0]
    bsz = 2048
    grid = (n // bsz,)
    rep = lambda i: (0, 0)
    out = pl.pallas_call(
        _fused_body,
        grid=grid,
        in_specs=[
            pl.BlockSpec((bsz, 2), lambda i: (i, 0)),
            pl.BlockSpec((_AXIS_RES, _EMB), rep),
            pl.BlockSpec((_AXIS_RES, _EMB), rep),
            pl.BlockSpec((_EMB, _HID), rep),
            pl.BlockSpec((1, _HID), rep),
            pl.BlockSpec((_HID, _HID), rep),
            pl.BlockSpec((1, _HID), rep),
            pl.BlockSpec((_HID, 3), rep),
            pl.BlockSpec((1, 3), rep),
        ],
        out_specs=pl.BlockSpec((bsz, 3), lambda i: (i, 0)),
        out_shape=jax.ShapeDtypeStruct((n, 3), jnp.float32),
    )(coords, emb0, emb1, W0.T, b0.reshape(1, -1), W1.T, b1.reshape(1, -1),
      W2.T, b2.reshape(1, -1))
    return out


# speedup vs baseline: 8.7555x; 2.1365x over previous
import jax
import jax.numpy as jnp
from jax.experimental import pallas as pl

_AXIS_RES = 512
_EMB = 256
_HID = 128
_FREQ = 30.0

_INV_PI = 0.3183098861837907
_PI_HI = 3.140625
_PI_LO = 9.676535897932385e-4
_S3 = -0.166666597127914428710938
_S5 = 0.00833307858556509017944336
_S7 = -0.000198106907191686332226
_S9 = 2.60831598097865935415e-06


def _fast_sin(x):
    nf = jnp.floor(x * _INV_PI + 0.5)
    r = x - nf * _PI_HI
    r = r - nf * _PI_LO
    r2 = r * r
    p = r + r * r2 * (_S3 + r2 * (_S5 + r2 * (_S7 + r2 * _S9)))
    odd = (nf.astype(jnp.int32) & 1) == 1
    return jnp.where(odd, -p, p)


def _fused_body(c_ref, e0_ref, e1_ref, w0_ref, b0_ref, w1_ref, b1_ref,
                w2_ref, b2_ref, o_ref):
    c = jnp.clip(c_ref[...], -1.0, 0.999)
    t = (0.5 * c + 0.5) * (_AXIS_RES - 1)
    bsz = c.shape[0]
    cols = jax.lax.broadcasted_iota(jnp.int32, (bsz, _AXIS_RES), 1).astype(jnp.float32)
    s0 = jnp.maximum(1.0 - jnp.abs(t[:, 0:1] - cols), 0.0)
    s1 = jnp.maximum(1.0 - jnp.abs(t[:, 1:2] - cols), 0.0)
    e0 = jnp.dot(s0, e0_ref[...], preferred_element_type=jnp.float32)
    e1 = jnp.dot(s1, e1_ref[...], preferred_element_type=jnp.float32)
    x = e0 * e1
    h = _fast_sin(_FREQ * (jnp.dot(x, w0_ref[...],
                                   preferred_element_type=jnp.float32) + b0_ref[...]))
    h = _fast_sin(_FREQ * (jnp.dot(h, w1_ref[...],
                                   preferred_element_type=jnp.float32) + b1_ref[...]))
    o_ref[...] = jnp.dot(h, w2_ref[...],
                         preferred_element_type=jnp.float32) + b2_ref[...]


def kernel(coords, emb0, emb1, W0, b0, W1, b1, W2, b2):
    n = coords.shape[0]
    bsz = 2048
    grid = (n // bsz,)
    rep = lambda i: (0, 0)
    out = pl.pallas_call(
        _fused_body,
        grid=grid,
        in_specs=[
            pl.BlockSpec((bsz, 2), lambda i: (i, 0)),
            pl.BlockSpec((_AXIS_RES, _EMB), rep),
            pl.BlockSpec((_AXIS_RES, _EMB), rep),
            pl.BlockSpec((_EMB, _HID), rep),
            pl.BlockSpec((1, _HID), rep),
            pl.BlockSpec((_HID, _HID), rep),
            pl.BlockSpec((1, _HID), rep),
            pl.BlockSpec((_HID, 3), rep),
            pl.BlockSpec((1, 3), rep),
        ],
        out_specs=pl.BlockSpec((bsz, 3), lambda i: (i, 0)),
        out_shape=jax.ShapeDtypeStruct((n, 3), jnp.float32),
    )(coords, emb0, emb1, W0.T, b0.reshape(1, -1), W1.T, b1.reshape(1, -1),
      W2.T, b2.reshape(1, -1))
    return out
